# Initial kernel scaffold; baseline (speedup 1.0000x reference)
import jax
import jax.numpy as jnp
from jax.experimental import pallas as pl


def kernel(sh_tensor, s_tensor, h_tensor, edge_index_SH, edge_index_SS, edge_index_HH, prescription, kgOneHot, p, params):
    raise NotImplementedError("write your pallas kernel here")


# speedup vs baseline: 17.3376x; 17.3376x over previous
import functools

import jax
import jax.numpy as jnp
from jax import lax
from jax.experimental import pallas as pl
from jax.experimental.pallas import tpu as pltpu
from jax.experimental.pallas import tpu_sc as plsc

_D = 512
_HEADS = 8
_N_SH = 1201
_N_S = 390
_N_H = 811
_E_SH = 80000
_E_SS = 20000
_E_HH = 40000
_B_PRESC = 1024

_NC = 2
_NS = 16
_L = 16


def _ceil_to(x, m):
    return (x + m - 1) // m * m


@functools.lru_cache(maxsize=None)
def _make_adj_builder(n, e, ch):
    rows = _ceil_to(_ceil_to(n, _NS) // _NS, 8)
    npad = _ceil_to(n, 8)
    e_half = e // _NC
    n_chunks = e_half // ch
    assert e_half % ch == 0 and ch % _L == 0

    mesh = plsc.VectorSubcoreMesh(core_axis_name="c", subcore_axis_name="s",
                                  num_cores=_NC, num_subcores=_NS)

    slab = rows * npad

    @functools.partial(
        pl.kernel,
        out_type=jax.ShapeDtypeStruct((_NC, _NS * slab), jnp.float32),
        mesh=mesh,
        scratch_types=[
            pltpu.VMEM((slab,), jnp.float32),
            pltpu.VMEM((ch,), jnp.int32),
            pltpu.VMEM((ch,), jnp.int32),
        ],
        compiler_params=pltpu.CompilerParams(use_tc_tiling_on_sc=False,
                                             needs_layout_passes=False),
    )
    def adj(src_hbm, dst_hbm, zeros_hbm, out_hbm, abuf, sbuf, dbuf):
        c = lax.axis_index("c")
        s = lax.axis_index("s")
        lo = s * rows
        pltpu.sync_copy(zeros_hbm, abuf)
        base = c * e_half
        ones = jnp.full((_L,), 1.0, jnp.float32)

        def chunk_body(k, carry):
            off = base + k * ch
            pltpu.sync_copy(src_hbm.at[pl.ds(off, ch)], sbuf)
            pltpu.sync_copy(dst_hbm.at[pl.ds(off, ch)], dbuf)

            def inner(i, carry2):
                sv = sbuf[pl.ds(i * _L, _L)]
                dv = dbuf[pl.ds(i * _L, _L)]
                r = dv - lo
                m = (r >= 0) & (r < rows)
                plsc.addupdate_scatter(abuf, [r * npad + sv], ones, mask=m)
                return carry2

            return lax.fori_loop(0, ch // _L, inner, carry)

        lax.fori_loop(0, n_chunks, chunk_body, 0)
        pltpu.sync_copy(abuf, out_hbm.at[c, pl.ds(s * slab, slab)])

    return adj


def _adj_sh(src, dst, zeros):
    return _make_adj_builder(_N_SH, _E_SH, 4000)(src, dst, zeros)


def _adj_ss(src, dst, zeros):
    return _make_adj_builder(_N_S, _E_SS, 2000)(src, dst, zeros)


def _adj_hh(src, dst, zeros):
    return _make_adj_builder(_N_H, _E_HH, 4000)(src, dst, zeros)


def _prep_adj(ap, n):
    def body(ap_ref, an_ref, rm_ref):
        a = (ap_ref[0] + ap_ref[1])[:n, :n]
        cnt = jnp.sum(a, axis=1)
        inv = 1.0 / jnp.maximum(cnt, 1.0)
        an_ref[...] = a * inv[:, None]
        rm_ref[...] = (cnt > 0).astype(jnp.float32)[:, None]

    return pl.pallas_call(body, out_shape=[
        jax.ShapeDtypeStruct((n, n), jnp.float32),
        jax.ShapeDtypeStruct((n, 1), jnp.float32),
    ])(ap)


def _tc_forward(a_sh, rm_sh, a_ss, rm_ss, a_hh, rm_hh, sh, s, h, kg, presc,
                hm, hmt, pvals, pkeys):
    n_in = 13 + len(pvals)

    def body(*refs):
        (ash_ref, rmsh_ref, ass_ref, rmss_ref, ahh_ref, rmhh_ref,
         sh_ref, s_ref, h_ref, kg_ref, presc_ref, hm_ref, hmt_ref) = refs[:13]
        prefs = refs[13:n_in]
        o_es, o_eh, o_sy = refs[n_in:]
        P = {k: prefs[i][...] for i, k in enumerate(pkeys)}
        f32 = jnp.float32

        def mmT(x, w):
            return lax.dot_general(x, w, (((1,), (1,)), ((), ())),
                                   preferred_element_type=f32)

        def mm(a, b):
            return lax.dot_general(a, b, (((1,), (0,)), ((), ())),
                                   preferred_element_type=f32)

        def lin(x, name):
            return mmT(x, P[name + "_w"]) + P[name + "_b"]

        def bn(x, name):
            m = jnp.mean(x, axis=0)
            xc = x - m
            v = jnp.mean(xc * xc, axis=0)
            return (xc / jnp.sqrt(v + 1e-5)) * P[name + "_g"] + P[name + "_be"]

        tanh = jnp.tanh

        A_sh, inv_sh, rm_sh = ash_ref[...], None, rmsh_ref[...]
        A_ss, inv_ss, rm_ss = ass_ref[...], None, rmss_ref[...]
        A_hh, inv_hh, rm_hh = ahh_ref[...], None, rmhh_ref[...]

        def gcn(x, name, A, inv, rm):
            y = mmT(x, P[name + "_w"])
            return tanh(mm(A, y) + P[name + "_b"] * rm)

        def mha(q, kv1, kv2, pre):
            Q = lin(q, pre + "_WQ")
            K1 = lin(kv1, pre + "_WK")
            K2 = lin(kv2, pre + "_WK")
            V1 = lin(kv1, pre + "_WV")
            V2 = lin(kv2, pre + "_WV")
            hmv = hm_ref[...]
            hmtv = hmt_ref[...]
            sc = 1.0 / jnp.sqrt(f32(256 // _HEADS))
            x1 = jnp.exp(mm(Q * K1, hmv) * sc)
            x2 = jnp.exp(mm(Q * K2, hmv) * sc)
            den = 1.0 + x1 + x2
            ctx = mm(x1 / den, hmtv) * V1 + mm(x2 / den, hmtv) * V2
            return lin(ctx, pre + "_fc")

        shv = sh_ref[...]
        sv = s_ref[...]
        hv = h_ref[...]
        kgv = kg_ref[...]

        esh0 = tanh(bn(lin(shv, "SH_s_mlp"), "SH_s_bn"))
        b0 = gcn(esh0, "convSH1", A_sh, inv_sh, rm_sh)
        b1 = tanh(bn(lin(esh0 + b0, "SH_line_s_1"), "SH_bn_s_1"))
        b1N = gcn(b1, "convSH2", A_sh, inv_sh, rm_sh)
        b2_sh = tanh(bn(lin(b1 + b1N, "SH_line_s_2"), "SH_bn_s_2"))

        esh02 = tanh(bn(lin(shv, "SH_h_mlp"), "SH_h_bn"))
        b0h = gcn(esh02, "convSH1h", A_sh, inv_sh, rm_sh)
        b1h = tanh(bn(lin(esh02 + b0h, "SH_line_h_1"), "SH_bn_h_1"))
        b1hN = gcn(b1h, "convSH2h", A_sh, inv_sh, rm_sh)
        b2_sh2 = tanh(bn(lin(b1h + b1hN, "SH_line_h_2"), "SH_bn_h_2"))

        es0 = tanh(bn(lin(sv, "SS_s_mlp"), "SS_s_bn"))
        r0 = gcn(es0, "convSS1", A_ss, inv_ss, rm_ss)
        r1s = tanh(bn(lin(es0 + r0, "SS_line_1"), "SS_bn_1"))
        r1N = gcn(r1s, "convSS2", A_ss, inv_ss, rm_ss)
        r2_s = tanh(bn(lin(r1s + r1N, "SS_line_2"), "SS_bn_2"))

        eh0 = tanh(bn(lin(hv, "HH_h_mlp"), "HH_h_bn"))
        kg0 = tanh(bn(lin(kgv, "kg_HH_mlp"), "kg_HH_bn"))
        eh0kg = eh0 + kg0
        rh0 = gcn(eh0kg, "convHH1", A_hh, inv_hh, rm_hh)
        r1h = tanh(bn(lin(eh0kg + rh0, "HH_line_1"), "HH_bn_1"))
        r1hN = gcn(r1h, "convHH2", A_hh, inv_hh, rm_hh)
        r2_h = tanh(bn(lin(r1h + r1hN, "HH_line_2"), "HH_bn_2"))

        es = mha(b2_sh[:_N_S] + r2_s, b2_sh[:_N_S], r2_s, "att_s")
        es = tanh(bn(es, "es_bn_1"))
        ehx = mha(b2_sh2[_N_S:] + r2_h, b2_sh2[_N_S:], r2_h, "att_h")
        ehx = tanh(bn(ehx, "eh_bn_1"))

        o_es[...] = es
        o_eh[...] = ehx
        o_sy[...] = mm(presc_ref[...], es)

    out_shape = [
        jax.ShapeDtypeStruct((_N_S, 256), jnp.float32),
        jax.ShapeDtypeStruct((_N_H, 256), jnp.float32),
        jax.ShapeDtypeStruct((_B_PRESC, 256), jnp.float32),
    ]
    return pl.pallas_call(body, out_shape=out_shape)(
        a_sh, rm_sh, a_ss, rm_ss, a_hh, rm_hh, sh, s, h, kg, presc,
        hm, hmt, *pvals)


def kernel(sh_tensor, s_tensor, h_tensor, edge_index_SH, edge_index_SS,
           edge_index_HH, prescription, kgOneHot, p, params):
    f32 = jnp.float32
    sh = jnp.asarray(sh_tensor, f32)
    s = jnp.asarray(s_tensor, f32)
    h = jnp.asarray(h_tensor, f32)
    presc = jnp.asarray(prescription, f32)
    kg = jnp.asarray(kgOneHot, f32)

    def adj(builder, ei, n, e):
        rows = _ceil_to(_ceil_to(n, _NS) // _NS, 8)
        npad = _ceil_to(n, 8)
        src = jnp.asarray(ei[0], jnp.int32)
        dst = jnp.asarray(ei[1], jnp.int32)
        zeros = jnp.zeros((rows * npad,), f32)
        return builder(src, dst, zeros).reshape(_NC, _NS * rows, npad)

    a_sh, rm_sh = _prep_adj(adj(_adj_sh, edge_index_SH, _N_SH, _E_SH), _N_SH)
    a_ss, rm_ss = _prep_adj(adj(_adj_ss, edge_index_SS, _N_S, _E_SS), _N_S)
    a_hh, rm_hh = _prep_adj(adj(_adj_hh, edge_index_HH, _N_H, _E_HH), _N_H)

    hm = jnp.repeat(jnp.eye(_HEADS, dtype=f32), 256 // _HEADS, axis=0)
    hmt = hm.T

    pkeys = tuple(sorted(params.keys()))
    pvals = [jnp.asarray(params[k], f32) for k in pkeys]

    es, ehx, e_synd = _tc_forward(a_sh, rm_sh, a_ss, rm_ss, a_hh, rm_hh,
                                  sh, s, h, kg, presc, hm, hmt, pvals, pkeys)
    out = jnp.concatenate([es, ehx, e_synd], axis=0)
    return out * jnp.asarray(p, out.dtype)


# speedup vs baseline: 20.9637x; 1.2091x over previous
import functools

import jax
import jax.numpy as jnp
from jax import lax
from jax.experimental import pallas as pl
from jax.experimental.pallas import tpu as pltpu
from jax.experimental.pallas import tpu_sc as plsc

_D = 512
_HEADS = 8
_N_SH = 1201
_N_S = 390
_N_H = 811
_E_SH = 80000
_E_SS = 20000
_E_HH = 40000
_B_PRESC = 1024

_NC = 2
_NS = 16
_L = 16


def _ceil_to(x, m):
    return (x + m - 1) // m * m


@functools.lru_cache(maxsize=None)
def _make_adj_builder(n, e, ch):
    rows = _ceil_to(_ceil_to(n, _NS) // _NS, 8)
    npad = _ceil_to(n, 8)
    e_half = e // _NC
    n_chunks = e_half // ch
    assert e_half % ch == 0 and ch % _L == 0

    mesh = plsc.VectorSubcoreMesh(core_axis_name="c", subcore_axis_name="s",
                                  num_cores=_NC, num_subcores=_NS)

    slab = rows * npad
    unroll = 5
    assert (ch // _L) % unroll == 0
    nbuf = 2 if n_chunks > 1 else 1

    @functools.partial(
        pl.kernel,
        out_type=jax.ShapeDtypeStruct((_NC, _NS * slab), jnp.float32),
        mesh=mesh,
        scratch_types=(
            [pltpu.VMEM((slab,), jnp.float32)]
            + [pltpu.VMEM((ch,), jnp.int32) for _ in range(2 * nbuf)]
            + [pltpu.SemaphoreType.DMA for _ in range(nbuf + 1)]
        ),
        compiler_params=pltpu.CompilerParams(use_tc_tiling_on_sc=False,
                                             needs_layout_passes=False),
    )
    def adj(src_hbm, dst_hbm, zeros_hbm, out_hbm, abuf, *rest):
        sbufs = rest[0:2 * nbuf:2]
        dbufs = rest[1:2 * nbuf:2]
        sems = rest[2 * nbuf:]
        c = lax.axis_index("c")
        s = lax.axis_index("s")
        lo = s * rows
        base = c * e_half
        ones = jnp.full((_L,), 1.0, jnp.float32)

        def start(k, b):
            off = base + k * ch
            return (pltpu.async_copy(src_hbm.at[pl.ds(off, ch)], sbufs[b], sems[b]),
                    pltpu.async_copy(dst_hbm.at[pl.ds(off, ch)], dbufs[b], sems[b]))

        zcp = pltpu.async_copy(zeros_hbm, abuf, sems[nbuf])
        pending = [None] * nbuf
        pending[0] = start(0, 0)
        zcp.wait()

        for k in range(n_chunks):
            b = k % nbuf
            if k + 1 < n_chunks:
                pending[(k + 1) % nbuf] = start(k + 1, (k + 1) % nbuf)
            h1, h2 = pending[b]
            h1.wait()
            h2.wait()
            sbuf, dbuf = sbufs[b], dbufs[b]

            def inner(i, carry, sbuf=sbuf, dbuf=dbuf):
                ib = i * (_L * unroll)
                for j in range(unroll):
                    sv = sbuf[pl.ds(ib + j * _L, _L)]
                    dv = dbuf[pl.ds(ib + j * _L, _L)]
                    r = dv - lo
                    m = (r >= 0) & (r < rows)
                    plsc.addupdate_scatter(abuf, [r * npad + sv], ones, mask=m)
                return carry

            lax.fori_loop(0, ch // (_L * unroll), inner, 0)

        pltpu.sync_copy(abuf, out_hbm.at[c, pl.ds(s * slab, slab)])

    return adj


def _adj_sh(src, dst, zeros):
    return _make_adj_builder(_N_SH, _E_SH, 4000)(src, dst, zeros)


def _adj_ss(src, dst, zeros):
    return _make_adj_builder(_N_S, _E_SS, 10000)(src, dst, zeros)


def _adj_hh(src, dst, zeros):
    return _make_adj_builder(_N_H, _E_HH, 20000)(src, dst, zeros)


def _prep_adj(ap, n):
    def body(ap_ref, an_ref, rm_ref):
        a = (ap_ref[0] + ap_ref[1])[:n, :n]
        cnt = jnp.sum(a, axis=1)
        inv = 1.0 / jnp.maximum(cnt, 1.0)
        an_ref[...] = a * inv[:, None]
        rm_ref[...] = (cnt > 0).astype(jnp.float32)[:, None]

    return pl.pallas_call(body, out_shape=[
        jax.ShapeDtypeStruct((n, n), jnp.float32),
        jax.ShapeDtypeStruct((n, 1), jnp.float32),
    ])(ap)


def _tc_forward(a_sh, rm_sh, a_ss, rm_ss, a_hh, rm_hh, sh, s, h, kg, presc,
                hm, hmt, pvals, pkeys):
    n_in = 13 + len(pvals)

    def body(*refs):
        (ash_ref, rmsh_ref, ass_ref, rmss_ref, ahh_ref, rmhh_ref,
         sh_ref, s_ref, h_ref, kg_ref, presc_ref, hm_ref, hmt_ref) = refs[:13]
        prefs = refs[13:n_in]
        o_es, o_eh, o_sy = refs[n_in:]
        P = {k: prefs[i][...] for i, k in enumerate(pkeys)}
        f32 = jnp.float32

        def mmT(x, w):
            return lax.dot_general(x, w, (((1,), (1,)), ((), ())),
                                   preferred_element_type=f32)

        def mm(a, b):
            return lax.dot_general(a, b, (((1,), (0,)), ((), ())),
                                   preferred_element_type=f32)

        def lin(x, name):
            return mmT(x, P[name + "_w"]) + P[name + "_b"]

        def bn(x, name):
            m = jnp.mean(x, axis=0)
            xc = x - m
            v = jnp.mean(xc * xc, axis=0)
            return (xc / jnp.sqrt(v + 1e-5)) * P[name + "_g"] + P[name + "_be"]

        tanh = jnp.tanh

        A_sh, inv_sh, rm_sh = ash_ref[...], None, rmsh_ref[...]
        A_ss, inv_ss, rm_ss = ass_ref[...], None, rmss_ref[...]
        A_hh, inv_hh, rm_hh = ahh_ref[...], None, rmhh_ref[...]

        def gcn(x, name, A, inv, rm):
            y = mmT(x, P[name + "_w"])
            return tanh(mm(A, y) + P[name + "_b"] * rm)

        def mha(q, kv1, kv2, pre):
            Q = lin(q, pre + "_WQ")
            K1 = lin(kv1, pre + "_WK")
            K2 = lin(kv2, pre + "_WK")
            V1 = lin(kv1, pre + "_WV")
            V2 = lin(kv2, pre + "_WV")
            hmv = hm_ref[...]
            hmtv = hmt_ref[...]
            sc = 1.0 / jnp.sqrt(f32(256 // _HEADS))
            x1 = jnp.exp(mm(Q * K1, hmv) * sc)
            x2 = jnp.exp(mm(Q * K2, hmv) * sc)
            den = 1.0 + x1 + x2
            ctx = mm(x1 / den, hmtv) * V1 + mm(x2 / den, hmtv) * V2
            return lin(ctx, pre + "_fc")

        shv = sh_ref[...]
        sv = s_ref[...]
        hv = h_ref[...]
        kgv = kg_ref[...]

        esh0 = tanh(bn(lin(shv, "SH_s_mlp"), "SH_s_bn"))
        b0 = gcn(esh0, "convSH1", A_sh, inv_sh, rm_sh)
        b1 = tanh(bn(lin(esh0 + b0, "SH_line_s_1"), "SH_bn_s_1"))
        b1N = gcn(b1, "convSH2", A_sh, inv_sh, rm_sh)
        b2_sh = tanh(bn(lin(b1 + b1N, "SH_line_s_2"), "SH_bn_s_2"))

        esh02 = tanh(bn(lin(shv, "SH_h_mlp"), "SH_h_bn"))
        b0h = gcn(esh02, "convSH1h", A_sh, inv_sh, rm_sh)
        b1h = tanh(bn(lin(esh02 + b0h, "SH_line_h_1"), "SH_bn_h_1"))
        b1hN = gcn(b1h, "convSH2h", A_sh, inv_sh, rm_sh)
        b2_sh2 = tanh(bn(lin(b1h + b1hN, "SH_line_h_2"), "SH_bn_h_2"))

        es0 = tanh(bn(lin(sv, "SS_s_mlp"), "SS_s_bn"))
        r0 = gcn(es0, "convSS1", A_ss, inv_ss, rm_ss)
        r1s = tanh(bn(lin(es0 + r0, "SS_line_1"), "SS_bn_1"))
        r1N = gcn(r1s, "convSS2", A_ss, inv_ss, rm_ss)
        r2_s = tanh(bn(lin(r1s + r1N, "SS_line_2"), "SS_bn_2"))

        eh0 = tanh(bn(lin(hv, "HH_h_mlp"), "HH_h_bn"))
        kg0 = tanh(bn(lin(kgv, "kg_HH_mlp"), "kg_HH_bn"))
        eh0kg = eh0 + kg0
        rh0 = gcn(eh0kg, "convHH1", A_hh, inv_hh, rm_hh)
        r1h = tanh(bn(lin(eh0kg + rh0, "HH_line_1"), "HH_bn_1"))
        r1hN = gcn(r1h, "convHH2", A_hh, inv_hh, rm_hh)
        r2_h = tanh(bn(lin(r1h + r1hN, "HH_line_2"), "HH_bn_2"))

        es = mha(b2_sh[:_N_S] + r2_s, b2_sh[:_N_S], r2_s, "att_s")
        es = tanh(bn(es, "es_bn_1"))
        ehx = mha(b2_sh2[_N_S:] + r2_h, b2_sh2[_N_S:], r2_h, "att_h")
        ehx = tanh(bn(ehx, "eh_bn_1"))

        o_es[...] = es
        o_eh[...] = ehx
        o_sy[...] = mm(presc_ref[...], es)

    out_shape = [
        jax.ShapeDtypeStruct((_N_S, 256), jnp.float32),
        jax.ShapeDtypeStruct((_N_H, 256), jnp.float32),
        jax.ShapeDtypeStruct((_B_PRESC, 256), jnp.float32),
    ]
    return pl.pallas_call(body, out_shape=out_shape)(
        a_sh, rm_sh, a_ss, rm_ss, a_hh, rm_hh, sh, s, h, kg, presc,
        hm, hmt, *pvals)


def kernel(sh_tensor, s_tensor, h_tensor, edge_index_SH, edge_index_SS,
           edge_index_HH, prescription, kgOneHot, p, params):
    f32 = jnp.float32
    sh = jnp.asarray(sh_tensor, f32)
    s = jnp.asarray(s_tensor, f32)
    h = jnp.asarray(h_tensor, f32)
    presc = jnp.asarray(prescription, f32)
    kg = jnp.asarray(kgOneHot, f32)

    def adj(builder, ei, n, e):
        rows = _ceil_to(_ceil_to(n, _NS) // _NS, 8)
        npad = _ceil_to(n, 8)
        src = jnp.asarray(ei[0], jnp.int32)
        dst = jnp.asarray(ei[1], jnp.int32)
        zeros = jnp.zeros((rows * npad,), f32)
        return builder(src, dst, zeros).reshape(_NC, _NS * rows, npad)

    a_sh, rm_sh = _prep_adj(adj(_adj_sh, edge_index_SH, _N_SH, _E_SH), _N_SH)
    a_ss, rm_ss = _prep_adj(adj(_adj_ss, edge_index_SS, _N_S, _E_SS), _N_S)
    a_hh, rm_hh = _prep_adj(adj(_adj_hh, edge_index_HH, _N_H, _E_HH), _N_H)

    hm = jnp.repeat(jnp.eye(_HEADS, dtype=f32), 256 // _HEADS, axis=0)
    hmt = hm.T

    pkeys = tuple(sorted(params.keys()))
    pvals = [jnp.asarray(params[k], f32) for k in pkeys]

    es, ehx, e_synd = _tc_forward(a_sh, rm_sh, a_ss, rm_ss, a_hh, rm_hh,
                                  sh, s, h, kg, presc, hm, hmt, pvals, pkeys)
    out = jnp.concatenate([es, ehx, e_synd], axis=0)
    return out * jnp.asarray(p, out.dtype)


# speedup vs baseline: 21.2899x; 1.0156x over previous
import functools

import jax
import jax.numpy as jnp
from jax import lax
from jax.experimental import pallas as pl
from jax.experimental.pallas import tpu as pltpu
from jax.experimental.pallas import tpu_sc as plsc

_D = 512
_HEADS = 8
_N_SH = 1201
_N_S = 390
_N_H = 811
_E_SH = 80000
_E_SS = 20000
_E_HH = 40000
_B_PRESC = 1024

_NC = 2
_NS = 16
_L = 16


def _ceil_to(x, m):
    return (x + m - 1) // m * m


@functools.lru_cache(maxsize=None)
def _make_adj_builder(n, e, ch):
    rows = _ceil_to(_ceil_to(n, _NS) // _NS, 8)
    npad = _ceil_to(n, 8)
    e_half = e // _NC
    n_chunks = e_half // ch
    assert e_half % ch == 0 and ch % _L == 0

    mesh = plsc.VectorSubcoreMesh(core_axis_name="c", subcore_axis_name="s",
                                  num_cores=_NC, num_subcores=_NS)

    slab = rows * npad
    unroll = 5
    assert (ch // _L) % unroll == 0
    nbuf = 2 if n_chunks > 1 else 1

    @functools.partial(
        pl.kernel,
        out_type=jax.ShapeDtypeStruct((_NC, _NS * slab), jnp.float32),
        mesh=mesh,
        scratch_types=(
            [pltpu.VMEM((slab,), jnp.float32)]
            + [pltpu.VMEM((ch,), jnp.int32) for _ in range(2 * nbuf)]
            + [pltpu.SemaphoreType.DMA for _ in range(nbuf + 1)]
        ),
        compiler_params=pltpu.CompilerParams(use_tc_tiling_on_sc=False,
                                             needs_layout_passes=False),
    )
    def adj(src_hbm, dst_hbm, zeros_hbm, out_hbm, abuf, *rest):
        sbufs = rest[0:2 * nbuf:2]
        dbufs = rest[1:2 * nbuf:2]
        sems = rest[2 * nbuf:]
        c = lax.axis_index("c")
        s = lax.axis_index("s")
        lo = s * rows
        base = c * e_half
        ones = jnp.full((_L,), 1.0, jnp.float32)

        def start(k, b):
            off = base + k * ch
            return (pltpu.async_copy(src_hbm.at[pl.ds(off, ch)], sbufs[b], sems[b]),
                    pltpu.async_copy(dst_hbm.at[pl.ds(off, ch)], dbufs[b], sems[b]))

        zcp = pltpu.async_copy(zeros_hbm, abuf, sems[nbuf])
        pending = [None] * nbuf
        pending[0] = start(0, 0)
        zcp.wait()

        for k in range(n_chunks):
            b = k % nbuf
            if k + 1 < n_chunks:
                pending[(k + 1) % nbuf] = start(k + 1, (k + 1) % nbuf)
            h1, h2 = pending[b]
            h1.wait()
            h2.wait()
            sbuf, dbuf = sbufs[b], dbufs[b]

            def inner(i, carry, sbuf=sbuf, dbuf=dbuf):
                ib = i * (_L * unroll)
                for j in range(unroll):
                    sv = sbuf[pl.ds(ib + j * _L, _L)]
                    dv = dbuf[pl.ds(ib + j * _L, _L)]
                    r = dv - lo
                    m = (r >= 0) & (r < rows)
                    plsc.addupdate_scatter(abuf, [r * npad + sv], ones, mask=m)
                return carry

            lax.fori_loop(0, ch // (_L * unroll), inner, 0)

        pltpu.sync_copy(abuf, out_hbm.at[c, pl.ds(s * slab, slab)])

    return adj


def _adj_sh(src, dst, zeros):
    return _make_adj_builder(_N_SH, _E_SH, 4000)(src, dst, zeros)


def _adj_ss(src, dst, zeros):
    return _make_adj_builder(_N_S, _E_SS, 10000)(src, dst, zeros)


def _adj_hh(src, dst, zeros):
    return _make_adj_builder(_N_H, _E_HH, 20000)(src, dst, zeros)


def _prep_adj(ap, n):
    def body(ap_ref, an_ref, rm_ref):
        a = (ap_ref[0] + ap_ref[1])[:n, :n]
        cnt = jnp.sum(a, axis=1)
        inv = 1.0 / jnp.maximum(cnt, 1.0)
        an_ref[...] = a * inv[:, None]
        rm_ref[...] = (cnt > 0).astype(jnp.float32)[:, None]

    return pl.pallas_call(body, out_shape=[
        jax.ShapeDtypeStruct((n, n), jnp.float32),
        jax.ShapeDtypeStruct((n, 1), jnp.float32),
    ])(ap)


def _mk_helpers(P):
    f32 = jnp.float32

    def mmT(x, w):
        return lax.dot_general(x, w, (((1,), (1,)), ((), ())),
                               preferred_element_type=f32)

    def mm(a, b):
        return lax.dot_general(a, b, (((1,), (0,)), ((), ())),
                               preferred_element_type=f32)

    def lin(x, name):
        return mmT(x, P[name + "_w"]) + P[name + "_b"]

    def bn(x, name):
        m = jnp.mean(x, axis=0)
        xc = x - m
        v = jnp.mean(xc * xc, axis=0)
        return (xc / jnp.sqrt(v + 1e-5)) * P[name + "_g"] + P[name + "_be"]

    return mmT, mm, lin, bn


_S1_NAMES = ("SH_s_mlp", "SH_s_bn", "SH_h_mlp", "SH_h_bn", "SS_s_mlp",
             "SS_s_bn", "HH_h_mlp", "HH_h_bn", "kg_HH_mlp", "kg_HH_bn")


def _tc_stage1(sh, s, h, kg, pvals, pkeys):
    n_in = 4 + len(pvals)

    def body(*refs):
        sh_ref, s_ref, h_ref, kg_ref = refs[:4]
        prefs = refs[4:n_in]
        o_esh0, o_esh02, o_es0, o_ehkg = refs[n_in:]
        P = {k: prefs[i][...] for i, k in enumerate(pkeys)}
        _, _, lin, bn = _mk_helpers(P)
        tanh = jnp.tanh
        o_esh0[...] = tanh(bn(lin(sh_ref[...], "SH_s_mlp"), "SH_s_bn"))
        o_esh02[...] = tanh(bn(lin(sh_ref[...], "SH_h_mlp"), "SH_h_bn"))
        o_es0[...] = tanh(bn(lin(s_ref[...], "SS_s_mlp"), "SS_s_bn"))
        eh0 = tanh(bn(lin(h_ref[...], "HH_h_mlp"), "HH_h_bn"))
        kg0 = tanh(bn(lin(kg_ref[...], "kg_HH_mlp"), "kg_HH_bn"))
        o_ehkg[...] = eh0 + kg0

    out_shape = [
        jax.ShapeDtypeStruct((_N_SH, _D), jnp.float32),
        jax.ShapeDtypeStruct((_N_SH, _D), jnp.float32),
        jax.ShapeDtypeStruct((_N_S, _D), jnp.float32),
        jax.ShapeDtypeStruct((_N_H, _D), jnp.float32),
    ]
    return pl.pallas_call(body, out_shape=out_shape)(sh, s, h, kg, *pvals)


def _tc_forward(a_sh, rm_sh, a_ss, rm_ss, a_hh, rm_hh, esh0_in, esh02_in,
                es0_in, ehkg_in, presc, hm, hmt, pvals, pkeys):
    n_in = 13 + len(pvals)

    def body(*refs):
        (ash_ref, rmsh_ref, ass_ref, rmss_ref, ahh_ref, rmhh_ref,
         esh0_ref, esh02_ref, es0_ref, ehkg_ref, presc_ref,
         hm_ref, hmt_ref) = refs[:13]
        prefs = refs[13:n_in]
        o_es, o_eh, o_sy = refs[n_in:]
        P = {k: prefs[i][...] for i, k in enumerate(pkeys)}
        f32 = jnp.float32
        mmT, mm, lin, bn = _mk_helpers(P)
        tanh = jnp.tanh

        A_sh, inv_sh, rm_sh = ash_ref[...], None, rmsh_ref[...]
        A_ss, inv_ss, rm_ss = ass_ref[...], None, rmss_ref[...]
        A_hh, inv_hh, rm_hh = ahh_ref[...], None, rmhh_ref[...]

        def gcn(x, name, A, inv, rm):
            y = mmT(x, P[name + "_w"])
            return tanh(mm(A, y) + P[name + "_b"] * rm)

        def mha(q, kv1, kv2, pre):
            Q = lin(q, pre + "_WQ")
            K1 = lin(kv1, pre + "_WK")
            K2 = lin(kv2, pre + "_WK")
            V1 = lin(kv1, pre + "_WV")
            V2 = lin(kv2, pre + "_WV")
            hmv = hm_ref[...]
            hmtv = hmt_ref[...]
            sc = 1.0 / jnp.sqrt(f32(256 // _HEADS))
            x1 = jnp.exp(mm(Q * K1, hmv) * sc)
            x2 = jnp.exp(mm(Q * K2, hmv) * sc)
            den = 1.0 + x1 + x2
            ctx = mm(x1 / den, hmtv) * V1 + mm(x2 / den, hmtv) * V2
            return lin(ctx, pre + "_fc")

        esh0 = esh0_ref[...]
        esh02 = esh02_ref[...]
        es0 = es0_ref[...]
        eh0kg = ehkg_ref[...]

        b0 = gcn(esh0, "convSH1", A_sh, inv_sh, rm_sh)
        b1 = tanh(bn(lin(esh0 + b0, "SH_line_s_1"), "SH_bn_s_1"))
        b1N = gcn(b1, "convSH2", A_sh, inv_sh, rm_sh)
        b2_sh = tanh(bn(lin(b1 + b1N, "SH_line_s_2"), "SH_bn_s_2"))

        b0h = gcn(esh02, "convSH1h", A_sh, inv_sh, rm_sh)
        b1h = tanh(bn(lin(esh02 + b0h, "SH_line_h_1"), "SH_bn_h_1"))
        b1hN = gcn(b1h, "convSH2h", A_sh, inv_sh, rm_sh)
        b2_sh2 = tanh(bn(lin(b1h + b1hN, "SH_line_h_2"), "SH_bn_h_2"))

        r0 = gcn(es0, "convSS1", A_ss, inv_ss, rm_ss)
        r1s = tanh(bn(lin(es0 + r0, "SS_line_1"), "SS_bn_1"))
        r1N = gcn(r1s, "convSS2", A_ss, inv_ss, rm_ss)
        r2_s = tanh(bn(lin(r1s + r1N, "SS_line_2"), "SS_bn_2"))

        rh0 = gcn(eh0kg, "convHH1", A_hh, inv_hh, rm_hh)
        r1h = tanh(bn(lin(eh0kg + rh0, "HH_line_1"), "HH_bn_1"))
        r1hN = gcn(r1h, "convHH2", A_hh, inv_hh, rm_hh)
        r2_h = tanh(bn(lin(r1h + r1hN, "HH_line_2"), "HH_bn_2"))

        es = mha(b2_sh[:_N_S] + r2_s, b2_sh[:_N_S], r2_s, "att_s")
        es = tanh(bn(es, "es_bn_1"))
        ehx = mha(b2_sh2[_N_S:] + r2_h, b2_sh2[_N_S:], r2_h, "att_h")
        ehx = tanh(bn(ehx, "eh_bn_1"))

        o_es[...] = es
        o_eh[...] = ehx
        o_sy[...] = mm(presc_ref[...], es)

    out_shape = [
        jax.ShapeDtypeStruct((_N_S, 256), jnp.float32),
        jax.ShapeDtypeStruct((_N_H, 256), jnp.float32),
        jax.ShapeDtypeStruct((_B_PRESC, 256), jnp.float32),
    ]
    return pl.pallas_call(body, out_shape=out_shape)(
        a_sh, rm_sh, a_ss, rm_ss, a_hh, rm_hh, esh0_in, esh02_in,
        es0_in, ehkg_in, presc, hm, hmt, *pvals)


def kernel(sh_tensor, s_tensor, h_tensor, edge_index_SH, edge_index_SS,
           edge_index_HH, prescription, kgOneHot, p, params):
    f32 = jnp.float32
    sh = jnp.asarray(sh_tensor, f32)
    s = jnp.asarray(s_tensor, f32)
    h = jnp.asarray(h_tensor, f32)
    presc = jnp.asarray(prescription, f32)
    kg = jnp.asarray(kgOneHot, f32)

    def adj(builder, ei, n, e):
        rows = _ceil_to(_ceil_to(n, _NS) // _NS, 8)
        npad = _ceil_to(n, 8)
        src = jnp.asarray(ei[0], jnp.int32)
        dst = jnp.asarray(ei[1], jnp.int32)
        zeros = jnp.zeros((rows * npad,), f32)
        return builder(src, dst, zeros).reshape(_NC, _NS * rows, npad)

    a_sh, rm_sh = _prep_adj(adj(_adj_sh, edge_index_SH, _N_SH, _E_SH), _N_SH)
    a_ss, rm_ss = _prep_adj(adj(_adj_ss, edge_index_SS, _N_S, _E_SS), _N_S)
    a_hh, rm_hh = _prep_adj(adj(_adj_hh, edge_index_HH, _N_H, _E_HH), _N_H)

    hm = jnp.repeat(jnp.eye(_HEADS, dtype=f32), 256 // _HEADS, axis=0)
    hmt = hm.T

    allkeys = tuple(sorted(params.keys()))
    s1keys = tuple(k for k in allkeys
                   if any(k.startswith(nm + "_") for nm in _S1_NAMES))
    s2keys = tuple(k for k in allkeys if k not in s1keys)
    s1vals = [jnp.asarray(params[k], f32) for k in s1keys]
    s2vals = [jnp.asarray(params[k], f32) for k in s2keys]

    esh0, esh02, es0, ehkg = _tc_stage1(sh, s, h, kg, s1vals, s1keys)

    es, ehx, e_synd = _tc_forward(a_sh, rm_sh, a_ss, rm_ss, a_hh, rm_hh,
                                  esh0, esh02, es0, ehkg, presc, hm, hmt,
                                  s2vals, s2keys)
    out = jnp.concatenate([es, ehx, e_synd], axis=0)
    return out * jnp.asarray(p, out.dtype)


# speedup vs baseline: 21.8668x; 1.0271x over previous
import functools

import jax
import jax.numpy as jnp
from jax import lax
from jax.experimental import pallas as pl
from jax.experimental.pallas import tpu as pltpu
from jax.experimental.pallas import tpu_sc as plsc

_D = 512
_HEADS = 8
_N_SH = 1201
_N_S = 390
_N_H = 811
_E_SH = 80000
_E_SS = 20000
_E_HH = 40000
_B_PRESC = 1024

_NC = 2
_NS = 16
_L = 16


def _ceil_to(x, m):
    return (x + m - 1) // m * m


@functools.lru_cache(maxsize=None)
def _make_adj_builder(n, e, ch):
    rows = _ceil_to(_ceil_to(n, _NS) // _NS, 8)
    npad = _ceil_to(n, 8)
    e_half = e // _NC
    n_chunks = e_half // ch
    assert e_half % ch == 0 and ch % _L == 0

    mesh = plsc.VectorSubcoreMesh(core_axis_name="c", subcore_axis_name="s",
                                  num_cores=_NC, num_subcores=_NS)

    slab = rows * npad
    unroll = 5
    assert (ch // _L) % unroll == 0
    nbuf = 2 if n_chunks > 1 else 1

    @functools.partial(
        pl.kernel,
        out_type=jax.ShapeDtypeStruct((_NC, _NS * slab), jnp.float32),
        mesh=mesh,
        scratch_types=(
            [pltpu.VMEM((slab,), jnp.float32)]
            + [pltpu.VMEM((ch,), jnp.int32) for _ in range(nbuf)]
            + [pltpu.SemaphoreType.DMA for _ in range(nbuf + 1)]
        ),
        compiler_params=pltpu.CompilerParams(use_tc_tiling_on_sc=False,
                                             needs_layout_passes=False),
    )
    def adj(packed_hbm, zeros_hbm, out_hbm, abuf, *rest):
        ebufs = rest[0:nbuf]
        sems = rest[nbuf:]
        c = lax.axis_index("c")
        s = lax.axis_index("s")
        losl = s * slab
        base = c * e_half
        ones = jnp.full((_L,), 1.0, jnp.float32)
        slab_u = jnp.uint32(slab)

        def start(k, b):
            off = base + k * ch
            return pltpu.async_copy(packed_hbm.at[pl.ds(off, ch)], ebufs[b], sems[b])

        zcp = pltpu.async_copy(zeros_hbm, abuf, sems[nbuf])
        pending = [None] * nbuf
        pending[0] = start(0, 0)
        zcp.wait()

        for k in range(n_chunks):
            b = k % nbuf
            if k + 1 < n_chunks:
                pending[(k + 1) % nbuf] = start(k + 1, (k + 1) % nbuf)
            pending[b].wait()
            ebuf = ebufs[b]

            def inner(i, carry, ebuf=ebuf):
                ib = i * (_L * unroll)
                for j in range(unroll):
                    pv = ebuf[pl.ds(ib + j * _L, _L)]
                    rf = pv - losl
                    m = lax.bitcast_convert_type(rf, jnp.uint32) < slab_u
                    plsc.addupdate_scatter(abuf, [rf], ones, mask=m)
                return carry

            lax.fori_loop(0, ch // (_L * unroll), inner, 0)

        pltpu.sync_copy(abuf, out_hbm.at[c, pl.ds(s * slab, slab)])

    return adj


def _adj_sh(packed, zeros):
    return _make_adj_builder(_N_SH, _E_SH, 8000)(packed, zeros)


def _adj_ss(packed, zeros):
    return _make_adj_builder(_N_S, _E_SS, 10000)(packed, zeros)


def _adj_hh(packed, zeros):
    return _make_adj_builder(_N_H, _E_HH, 20000)(packed, zeros)


def _prep_adj(ap, n):
    def body(ap_ref, an_ref, rm_ref):
        a = (ap_ref[0] + ap_ref[1])[:n, :n]
        cnt = jnp.sum(a, axis=1)
        inv = 1.0 / jnp.maximum(cnt, 1.0)
        an_ref[...] = a * inv[:, None]
        rm_ref[...] = (cnt > 0).astype(jnp.float32)[:, None]

    return pl.pallas_call(body, out_shape=[
        jax.ShapeDtypeStruct((n, n), jnp.float32),
        jax.ShapeDtypeStruct((n, 1), jnp.float32),
    ])(ap)


def _mk_helpers(P):
    f32 = jnp.float32

    def mmT(x, w):
        return lax.dot_general(x, w, (((1,), (1,)), ((), ())),
                               preferred_element_type=f32)

    def mm(a, b):
        return lax.dot_general(a, b, (((1,), (0,)), ((), ())),
                               preferred_element_type=f32)

    def lin(x, name):
        return mmT(x, P[name + "_w"]) + P[name + "_b"]

    def bn(x, name):
        m = jnp.mean(x, axis=0)
        xc = x - m
        v = jnp.mean(xc * xc, axis=0)
        return (xc / jnp.sqrt(v + 1e-5)) * P[name + "_g"] + P[name + "_be"]

    return mmT, mm, lin, bn


_S1_NAMES = ("SH_s_mlp", "SH_s_bn", "SH_h_mlp", "SH_h_bn", "SS_s_mlp",
             "SS_s_bn", "HH_h_mlp", "HH_h_bn", "kg_HH_mlp", "kg_HH_bn")


def _tc_stage1(sh, s, h, kg, pvals, pkeys):
    n_in = 4 + len(pvals)

    def body(*refs):
        sh_ref, s_ref, h_ref, kg_ref = refs[:4]
        prefs = refs[4:n_in]
        o_esh0, o_esh02, o_es0, o_ehkg = refs[n_in:]
        P = {k: prefs[i][...] for i, k in enumerate(pkeys)}
        _, _, lin, bn = _mk_helpers(P)
        tanh = jnp.tanh
        o_esh0[...] = tanh(bn(lin(sh_ref[...], "SH_s_mlp"), "SH_s_bn"))
        o_esh02[...] = tanh(bn(lin(sh_ref[...], "SH_h_mlp"), "SH_h_bn"))
        o_es0[...] = tanh(bn(lin(s_ref[...], "SS_s_mlp"), "SS_s_bn"))
        eh0 = tanh(bn(lin(h_ref[...], "HH_h_mlp"), "HH_h_bn"))
        kg0 = tanh(bn(lin(kg_ref[...], "kg_HH_mlp"), "kg_HH_bn"))
        o_ehkg[...] = eh0 + kg0

    out_shape = [
        jax.ShapeDtypeStruct((_N_SH, _D), jnp.float32),
        jax.ShapeDtypeStruct((_N_SH, _D), jnp.float32),
        jax.ShapeDtypeStruct((_N_S, _D), jnp.float32),
        jax.ShapeDtypeStruct((_N_H, _D), jnp.float32),
    ]
    return pl.pallas_call(body, out_shape=out_shape)(sh, s, h, kg, *pvals)


def _tc_forward(a_sh, rm_sh, a_ss, rm_ss, a_hh, rm_hh, esh0_in, esh02_in,
                es0_in, ehkg_in, presc, hm, hmt, pvals, pkeys):
    n_in = 13 + len(pvals)

    def body(*refs):
        (ash_ref, rmsh_ref, ass_ref, rmss_ref, ahh_ref, rmhh_ref,
         esh0_ref, esh02_ref, es0_ref, ehkg_ref, presc_ref,
         hm_ref, hmt_ref) = refs[:13]
        prefs = refs[13:n_in]
        o_es, o_eh, o_sy = refs[n_in:]
        P = {k: prefs[i][...] for i, k in enumerate(pkeys)}
        f32 = jnp.float32
        mmT, mm, lin, bn = _mk_helpers(P)
        tanh = jnp.tanh

        A_sh, inv_sh, rm_sh = ash_ref[...], None, rmsh_ref[...]
        A_ss, inv_ss, rm_ss = ass_ref[...], None, rmss_ref[...]
        A_hh, inv_hh, rm_hh = ahh_ref[...], None, rmhh_ref[...]

        def gcn(x, name, A, inv, rm):
            y = mmT(x, P[name + "_w"])
            return tanh(mm(A, y) + P[name + "_b"] * rm)

        def mha(q, kv1, kv2, pre):
            Q = lin(q, pre + "_WQ")
            K1 = lin(kv1, pre + "_WK")
            K2 = lin(kv2, pre + "_WK")
            V1 = lin(kv1, pre + "_WV")
            V2 = lin(kv2, pre + "_WV")
            hmv = hm_ref[...]
            hmtv = hmt_ref[...]
            sc = 1.0 / jnp.sqrt(f32(256 // _HEADS))
            x1 = jnp.exp(mm(Q * K1, hmv) * sc)
            x2 = jnp.exp(mm(Q * K2, hmv) * sc)
            den = 1.0 + x1 + x2
            ctx = mm(x1 / den, hmtv) * V1 + mm(x2 / den, hmtv) * V2
            return lin(ctx, pre + "_fc")

        esh0 = esh0_ref[...]
        esh02 = esh02_ref[...]
        es0 = es0_ref[...]
        eh0kg = ehkg_ref[...]

        b0 = gcn(esh0, "convSH1", A_sh, inv_sh, rm_sh)
        b1 = tanh(bn(lin(esh0 + b0, "SH_line_s_1"), "SH_bn_s_1"))
        b1N = gcn(b1, "convSH2", A_sh, inv_sh, rm_sh)
        b2_sh = tanh(bn(lin(b1 + b1N, "SH_line_s_2"), "SH_bn_s_2"))

        b0h = gcn(esh02, "convSH1h", A_sh, inv_sh, rm_sh)
        b1h = tanh(bn(lin(esh02 + b0h, "SH_line_h_1"), "SH_bn_h_1"))
        b1hN = gcn(b1h, "convSH2h", A_sh, inv_sh, rm_sh)
        b2_sh2 = tanh(bn(lin(b1h + b1hN, "SH_line_h_2"), "SH_bn_h_2"))

        r0 = gcn(es0, "convSS1", A_ss, inv_ss, rm_ss)
        r1s = tanh(bn(lin(es0 + r0, "SS_line_1"), "SS_bn_1"))
        r1N = gcn(r1s, "convSS2", A_ss, inv_ss, rm_ss)
        r2_s = tanh(bn(lin(r1s + r1N, "SS_line_2"), "SS_bn_2"))

        rh0 = gcn(eh0kg, "convHH1", A_hh, inv_hh, rm_hh)
        r1h = tanh(bn(lin(eh0kg + rh0, "HH_line_1"), "HH_bn_1"))
        r1hN = gcn(r1h, "convHH2", A_hh, inv_hh, rm_hh)
        r2_h = tanh(bn(lin(r1h + r1hN, "HH_line_2"), "HH_bn_2"))

        es = mha(b2_sh[:_N_S] + r2_s, b2_sh[:_N_S], r2_s, "att_s")
        es = tanh(bn(es, "es_bn_1"))
        ehx = mha(b2_sh2[_N_S:] + r2_h, b2_sh2[_N_S:], r2_h, "att_h")
        ehx = tanh(bn(ehx, "eh_bn_1"))

        o_es[...] = es
        o_eh[...] = ehx
        o_sy[...] = mm(presc_ref[...], es)

    out_shape = [
        jax.ShapeDtypeStruct((_N_S, 256), jnp.float32),
        jax.ShapeDtypeStruct((_N_H, 256), jnp.float32),
        jax.ShapeDtypeStruct((_B_PRESC, 256), jnp.float32),
    ]
    return pl.pallas_call(body, out_shape=out_shape)(
        a_sh, rm_sh, a_ss, rm_ss, a_hh, rm_hh, esh0_in, esh02_in,
        es0_in, ehkg_in, presc, hm, hmt, *pvals)


def kernel(sh_tensor, s_tensor, h_tensor, edge_index_SH, edge_index_SS,
           edge_index_HH, prescription, kgOneHot, p, params):
    f32 = jnp.float32
    sh = jnp.asarray(sh_tensor, f32)
    s = jnp.asarray(s_tensor, f32)
    h = jnp.asarray(h_tensor, f32)
    presc = jnp.asarray(prescription, f32)
    kg = jnp.asarray(kgOneHot, f32)

    def adj(builder, ei, n, e):
        rows = _ceil_to(_ceil_to(n, _NS) // _NS, 8)
        npad = _ceil_to(n, 8)
        src = jnp.asarray(ei[0], jnp.int32)
        dst = jnp.asarray(ei[1], jnp.int32)
        packed = dst * npad + src
        zeros = jnp.zeros((rows * npad,), f32)
        return builder(packed, zeros).reshape(_NC, _NS * rows, npad)

    a_sh, rm_sh = _prep_adj(adj(_adj_sh, edge_index_SH, _N_SH, _E_SH), _N_SH)
    a_ss, rm_ss = _prep_adj(adj(_adj_ss, edge_index_SS, _N_S, _E_SS), _N_S)
    a_hh, rm_hh = _prep_adj(adj(_adj_hh, edge_index_HH, _N_H, _E_HH), _N_H)

    hm = jnp.repeat(jnp.eye(_HEADS, dtype=f32), 256 // _HEADS, axis=0)
    hmt = hm.T

    allkeys = tuple(sorted(params.keys()))
    s1keys = tuple(k for k in allkeys
                   if any(k.startswith(nm + "_") for nm in _S1_NAMES))
    s2keys = tuple(k for k in allkeys if k not in s1keys)
    s1vals = [jnp.asarray(params[k], f32) for k in s1keys]
    s2vals = [jnp.asarray(params[k], f32) for k in s2keys]

    esh0, esh02, es0, ehkg = _tc_stage1(sh, s, h, kg, s1vals, s1keys)

    es, ehx, e_synd = _tc_forward(a_sh, rm_sh, a_ss, rm_ss, a_hh, rm_hh,
                                  esh0, esh02, es0, ehkg, presc, hm, hmt,
                                  s2vals, s2keys)
    out = jnp.concatenate([es, ehx, e_synd], axis=0)
    return out * jnp.asarray(p, out.dtype)
